# Initial kernel scaffold; baseline (speedup 1.0000x reference)
import jax
import jax.numpy as jnp
from jax.experimental import pallas as pl


def kernel(x, edge_index, W0, b0, W1, b1, W2, b2):
    raise NotImplementedError("write your pallas kernel here")


# speedup vs baseline: 8.0247x; 8.0247x over previous
import functools

import jax
import jax.numpy as jnp
from jax import lax
from jax.experimental import pallas as pl
from jax.experimental.pallas import tpu as pltpu
from jax.experimental.pallas import tpu_sc as plsc

N = 10000
NP = 10240
E = 320000
NW = 32
EW = 10240
EP = NW * EW
RT = NP // 16
BR = 1280
GRID = NP // BR

_MESH = plsc.VectorSubcoreMesh(core_axis_name="c", subcore_axis_name="s")


def _make_agg(D, CH):
    K = EW // CH

    @functools.partial(
        pl.kernel,
        out_type=jax.ShapeDtypeStruct((2, NP, D), jnp.float32),
        mesh=_MESH,
        scratch_types=[
            pltpu.VMEM_SHARED((NP, D), jnp.float32),
            pltpu.VMEM((EW,), jnp.int32),
            pltpu.VMEM((K, CH), jnp.int32),
            pltpu.VMEM((CH, D), jnp.float32),
            pltpu.VMEM((CH, D), jnp.float32),
            pltpu.SemaphoreType.DMA,
            pltpu.SemaphoreType.DMA,
        ],
    )
    def agg(hs, zeros, srcp, dstp, out, acc, srcv, dstv, buf0, buf1, sem0, sem1):
        c = lax.axis_index("c")
        s = lax.axis_index("s")
        w = c * 16 + s
        rows = pl.ds(s * RT, RT)

        @pl.when(c == 0)
        def _():
            pltpu.sync_copy(hs.at[rows], acc.at[rows])

        @pl.when(c != 0)
        def _():
            pltpu.sync_copy(zeros.at[rows], acc.at[rows])

        pltpu.sync_copy(srcp.at[w], srcv)
        pltpu.sync_copy(dstp.at[w], dstv)
        plsc.subcore_barrier()

        @pl.loop(0, K, step=2)
        def _(j):
            d0 = pltpu.async_copy(hs.at[srcv.at[pl.ds(j * CH, CH)]], buf0, sem0)
            d1 = pltpu.async_copy(
                hs.at[srcv.at[pl.ds((j + 1) * CH, CH)]], buf1, sem1)
            d0.wait()
            pltpu.sync_copy(buf0, acc.at[dstv.at[j]], add=True)
            d1.wait()
            pltpu.sync_copy(buf1, acc.at[dstv.at[j + 1]], add=True)

        plsc.subcore_barrier()
        pltpu.sync_copy(acc.at[rows], out.at[c, rows])

    return agg


_DEG_CH = 128
_DEG_K = EW // _DEG_CH


@functools.partial(
    pl.kernel,
    out_type=jax.ShapeDtypeStruct((2, NP), jnp.float32),
    mesh=_MESH,
    scratch_types=[
        pltpu.VMEM_SHARED((NP,), jnp.float32),
        pltpu.VMEM((_DEG_K, _DEG_CH), jnp.int32),
        pltpu.VMEM((_DEG_CH,), jnp.float32),
    ],
)
def _deg(zeros1, dstp, out, acc, dstv, ones):
    c = lax.axis_index("c")
    s = lax.axis_index("s")
    w = c * 16 + s
    rows = pl.ds(s * RT, RT)
    pltpu.sync_copy(zeros1.at[rows], acc.at[rows])
    pltpu.sync_copy(dstp.at[w], dstv)

    @pl.loop(0, _DEG_CH // 16)
    def _(i):
        ones[pl.ds(i * 16, 16)] = jnp.ones((16,), jnp.float32)

    plsc.subcore_barrier()

    @pl.loop(0, _DEG_K)
    def _(j):
        pltpu.sync_copy(ones, acc.at[dstv.at[j]], add=True)

    plsc.subcore_barrier()
    pltpu.sync_copy(acc.at[rows], out.at[c, rows])


def _tc1_body(cnt_ref, x_ref, w_ref, dinv_ref, hs_ref):
    dinv = lax.rsqrt(cnt_ref[0] + cnt_ref[1] + 1.0)
    dinv_ref[...] = dinv
    h = jnp.dot(x_ref[...], w_ref[...], preferred_element_type=jnp.float32)
    hs_ref[...] = h * dinv


def _tc1(cnt, x, W0):
    return pl.pallas_call(
        _tc1_body,
        grid=(GRID,),
        in_specs=[
            pl.BlockSpec((2, BR, 1), lambda i: (0, i, 0)),
            pl.BlockSpec((BR, 128), lambda i: (i, 0)),
            pl.BlockSpec((128, 128), lambda i: (0, 0)),
        ],
        out_specs=[
            pl.BlockSpec((BR, 1), lambda i: (i, 0)),
            pl.BlockSpec((BR, 128), lambda i: (i, 0)),
        ],
        out_shape=[
            jax.ShapeDtypeStruct((NP, 1), jnp.float32),
            jax.ShapeDtypeStruct((NP, 128), jnp.float32),
        ],
    )(cnt, x, W0)


def _tcmid_body(p_ref, dinv_ref, b_ref, w_ref, hs_ref):
    dinv = dinv_ref[...]
    z = jnp.maximum((p_ref[0] + p_ref[1]) * dinv + b_ref[...], 0.0)
    h = jnp.dot(z, w_ref[...], preferred_element_type=jnp.float32)
    hs_ref[...] = h * dinv


def _tcmid(p, dinv, b, W, dout):
    din = p.shape[-1]
    return pl.pallas_call(
        _tcmid_body,
        grid=(GRID,),
        in_specs=[
            pl.BlockSpec((2, BR, din), lambda i: (0, i, 0)),
            pl.BlockSpec((BR, 1), lambda i: (i, 0)),
            pl.BlockSpec((1, din), lambda i: (0, 0)),
            pl.BlockSpec((din, dout), lambda i: (0, 0)),
        ],
        out_specs=pl.BlockSpec((BR, dout), lambda i: (i, 0)),
        out_shape=jax.ShapeDtypeStruct((NP, dout), jnp.float32),
    )(p, dinv, b.reshape(1, din), W)


def _tc4_body(p_ref, dinv_ref, b_ref, out_ref):
    zf = (p_ref[0] + p_ref[1]) * dinv_ref[...]
    z = zf[:, :64] + b_ref[...]
    m = jnp.max(z, axis=-1, keepdims=True)
    lse = jnp.log(jnp.sum(jnp.exp(z - m), axis=-1, keepdims=True)) + m
    out_ref[...] = z - lse


def _tc4(p, dinv, b):
    return pl.pallas_call(
        _tc4_body,
        grid=(GRID,),
        in_specs=[
            pl.BlockSpec((2, BR, 128), lambda i: (0, i, 0)),
            pl.BlockSpec((BR, 1), lambda i: (i, 0)),
            pl.BlockSpec((1, 64), lambda i: (0, 0)),
        ],
        out_specs=pl.BlockSpec((BR, 64), lambda i: (i, 0)),
        out_shape=jax.ShapeDtypeStruct((NP, 64), jnp.float32),
    )(p, dinv, b.reshape(1, 64))


_agg128 = _make_agg(128, 64)


def kernel(x, edge_index, W0, b0, W1, b1, W2, b2):
    pad = jnp.full((EP - E,), N, jnp.int32)
    src = jnp.concatenate([edge_index[0], pad]).reshape(NW, EW)
    dst = jnp.concatenate([edge_index[1], pad])
    dst64 = dst.reshape(NW, EW // 64, 64)
    dst128 = dst.reshape(NW, EW // 128, 128)
    xp = jnp.pad(x, ((0, NP - N), (0, 0)))
    W2p = jnp.pad(W2, ((0, 0), (0, 64)))
    zeros128 = jnp.zeros((NP, 128), jnp.float32)
    zeros1 = jnp.zeros((NP,), jnp.float32)

    cnt = _deg(zeros1, dst128)
    dinv, hs0 = _tc1(cnt.reshape(2, NP, 1), xp, W0)
    p = _agg128(hs0, zeros128, src, dst64)
    hs1 = _tcmid(p, dinv, b0, W1, 128)
    p = _agg128(hs1, zeros128, src, dst64)
    hs2 = _tcmid(p, dinv, b1, W2p, 128)
    q = _agg128(hs2, zeros128, src, dst64)
    return _tc4(q, dinv, b2)[:N]
